# baseline (speedup 1.0000x reference)
import functools
import math

import jax
import jax.numpy as jnp
from jax import lax
from jax.experimental import pallas as pl
from jax.experimental.pallas import tpu as pltpu
from jax.experimental.pallas import tpu_sc as plsc

D_EMBED = 128
SCALE = math.sqrt(float(D_EMBED))
CHUNK = 128
LANES = 16
NBUF = 3


@functools.lru_cache(maxsize=None)
def _build(seq, batch):
    info = plsc.get_sparse_core_info()
    NC, NS = info.num_cores, info.num_subcores
    NW = NC * NS
    assert batch % (NW * CHUNK) == 0 and seq % 2 == 0
    n_pairs = seq // 2
    n_outer = -(-n_pairs // NBUF)
    mesh = plsc.VectorSubcoreMesh(core_axis_name="c", subcore_axis_name="s")

    @functools.partial(
        pl.kernel,
        mesh=mesh,
        out_type=jax.ShapeDtypeStruct((seq, batch, D_EMBED), jnp.float32),
        scratch_types=(
            [
                pltpu.VMEM((seq, CHUNK), jnp.int32),
                pltpu.VMEM((NBUF, 2, CHUNK, D_EMBED), jnp.float32),
            ]
            + [pltpu.SemaphoreType.DMA] * (2 * NBUF)
        ),
    )
    def gather_scale(xt_hbm, table_hbm, out_hbm, idx_all, rows_v, *sems):
        gsem = sems[:NBUF]
        wsem = sems[NBUF:]
        cid = lax.axis_index("c")
        sid = lax.axis_index("s")
        wid = sid * NC + cid
        col0 = wid * CHUNK

        pltpu.sync_copy(xt_hbm.at[pl.ds(0, seq), pl.ds(col0, CHUNK)], idx_all)

        def issue_gather(p, b):
            for k in range(2):
                pltpu.async_copy(
                    table_hbm.at[idx_all.at[2 * p + k]], rows_v.at[b, k], gsem[b]
                )

        def wait_gather(b):
            for k in range(2):
                pltpu.make_async_copy(
                    table_hbm.at[idx_all.at[0]], rows_v.at[b, k], gsem[b]
                ).wait()

        def issue_write(p, b):
            pltpu.async_copy(
                rows_v.at[b],
                out_hbm.at[pl.ds(2 * p, 2), pl.ds(col0, CHUNK)],
                wsem[b],
            )

        def wait_write(b):
            pltpu.make_async_copy(
                rows_v.at[b], out_hbm.at[pl.ds(0, 2), pl.ds(col0, CHUNK)], wsem[b]
            ).wait()

        def scale(b):
            def row_body(r, c):
                for k in range(2):
                    for j in range(D_EMBED // LANES):
                        sl = pl.ds(j * LANES, LANES)
                        rows_v[b, k, r, sl] = rows_v[b, k, r, sl] * SCALE
                return c

            lax.fori_loop(0, CHUNK, row_body, 0)

        for b in range(NBUF):
            issue_gather(b, b)

        def outer(g, c):
            for b in range(NBUF):
                p = g * NBUF + b

                @pl.when(p < n_pairs)
                def _():
                    wait_gather(b)
                    scale(b)
                    issue_write(p, b)

            for b in range(NBUF):
                p = g * NBUF + b

                @pl.when(p < n_pairs)
                def _():
                    wait_write(b)

                @pl.when(p + NBUF < n_pairs)
                def _():
                    issue_gather(p + NBUF, b)

            return c

        lax.fori_loop(0, n_outer, outer, 0)

    return gather_scale


def kernel(x, table):
    batch, seq = x.shape
    xt = x.T.astype(jnp.int32)
    o = _build(seq, batch)(xt, table)
    return o.transpose(1, 0, 2)


# speedup vs baseline: 1.0334x; 1.0334x over previous
import functools
import math

import jax
import jax.numpy as jnp
from jax import lax
from jax.experimental import pallas as pl
from jax.experimental.pallas import tpu as pltpu
from jax.experimental.pallas import tpu_sc as plsc

D_EMBED = 128
SCALE = math.sqrt(float(D_EMBED))
CHUNK = 128
LANES = 16
NBUF = 5


@functools.lru_cache(maxsize=None)
def _build(seq, batch):
    info = plsc.get_sparse_core_info()
    NC, NS = info.num_cores, info.num_subcores
    NW = NC * NS
    assert batch % (NW * CHUNK) == 0 and seq % NBUF == 0
    n_outer = seq // NBUF
    mesh = plsc.VectorSubcoreMesh(core_axis_name="c", subcore_axis_name="s")

    @functools.partial(
        pl.kernel,
        mesh=mesh,
        out_type=jax.ShapeDtypeStruct((seq, batch, D_EMBED), jnp.float32),
        scratch_types=(
            [
                pltpu.VMEM((seq, CHUNK), jnp.int32),
                pltpu.VMEM((NBUF, CHUNK, D_EMBED), jnp.float32),
            ]
            + [pltpu.SemaphoreType.DMA] * (2 * NBUF)
        ),
    )
    def gather_scale(xt_hbm, table_hbm, out_hbm, idx_all, rows_v, *sems):
        gsem = sems[:NBUF]
        wsem = sems[NBUF:]
        cid = lax.axis_index("c")
        sid = lax.axis_index("s")
        wid = sid * NC + cid
        col0 = wid * CHUNK

        pltpu.sync_copy(xt_hbm.at[pl.ds(0, seq), pl.ds(col0, CHUNK)], idx_all)

        def issue_gather(s, b):
            pltpu.async_copy(table_hbm.at[idx_all.at[s]], rows_v.at[b], gsem[b])

        def wait_gather(b):
            pltpu.make_async_copy(
                table_hbm.at[idx_all.at[0]], rows_v.at[b], gsem[b]
            ).wait()

        def issue_write(s, b):
            pltpu.async_copy(
                rows_v.at[b], out_hbm.at[s, pl.ds(col0, CHUNK)], wsem[b]
            )

        def wait_write(b):
            pltpu.make_async_copy(
                rows_v.at[b], out_hbm.at[0, pl.ds(col0, CHUNK)], wsem[b]
            ).wait()

        def scale(b):
            def row_body(r, c):
                def j_body(j, c2):
                    sl = pl.ds(j * LANES, LANES)
                    rows_v[b, r, sl] = rows_v[b, r, sl] * SCALE
                    return c2

                return lax.fori_loop(0, D_EMBED // LANES, j_body, c)

            lax.fori_loop(0, CHUNK, row_body, 0)

        for b in range(NBUF):
            issue_gather(b, b)

        def outer(g, c):
            for b in range(NBUF):
                wait_gather(b)
                scale(b)
                issue_write(g * NBUF + b, b)
            for b in range(NBUF):
                wait_write(b)

                @pl.when(g < n_outer - 1)
                def _():
                    issue_gather((g + 1) * NBUF + b, b)

            return c

        lax.fori_loop(0, n_outer, outer, 0)

    return gather_scale


def kernel(x, table):
    batch, seq = x.shape
    xt = x.T.astype(jnp.int32)
    o = _build(seq, batch)(xt, table)
    return o.transpose(1, 0, 2)
